# baseline (speedup 1.0000x reference)
import jax
import jax.numpy as jnp
from jax import lax
from jax.experimental import pallas as pl
from jax.experimental.pallas import tpu as pltpu

_B, _T, _V = 16, 32, 100000
_G = 4
_NG = _B // _G


def _body(s_smem, r_smem, b_smem, probs_hbm, s_hbm, o_ref, buf, pv_ref, sv_buf,
          sems, ssem):
    scp = pltpu.make_async_copy(s_hbm, sv_buf, ssem)
    scp.start()
    for b in range(_B):
        for t in range(_T):
            w = pl.multiple_of((s_smem[b, t] >> 7) << 7, 128)
            pltpu.make_async_copy(
                probs_hbm.at[b, t, pl.ds(w, 128)], buf.at[b, t],
                sems.at[b // _G],
            ).start()

    scp.wait()
    sv = sv_buf[...]
    lane3 = jnp.broadcast_to((sv & 127)[:, :, None], (_B, _T, 128))
    oh = lax.broadcasted_iota(jnp.int32, (_B, _T, 128), 2) == lane3
    seq_len = jnp.sum((sv > 0).astype(jnp.int32), axis=1, keepdims=True) + 1
    tt = lax.broadcasted_iota(jnp.int32, (_B, _T), 1)
    maskv = (tt < seq_len).astype(jnp.float32)

    for g in range(_NG):
        rows = pl.ds(g * _G, _G)
        pltpu.make_async_copy(
            probs_hbm.at[rows, pl.ds(0, _T), pl.ds(0, 128)],
            buf.at[rows], sems.at[g],
        ).wait()
        pv_ref[rows] = jnp.sum(
            jnp.where(oh[g * _G:(g + 1) * _G], buf[g * _G:(g + 1) * _G], 0.0),
            axis=2,
        )

    lrows = -jnp.log(pv_ref[...] + 1e-10) * maskv
    acc = 0.0
    for b in range(_B):
        acc += (r_smem[b] - b_smem[b]) * jnp.sum(lrows[b])
    o_ref[0, 0] = acc / jnp.sum(maskv)


def kernel(reward, baseline, probs, seqs):
    seqs = seqs.astype(jnp.int32)
    out = pl.pallas_call(
        _body,
        in_specs=[
            pl.BlockSpec(memory_space=pltpu.MemorySpace.SMEM),
            pl.BlockSpec(memory_space=pltpu.MemorySpace.SMEM),
            pl.BlockSpec(memory_space=pltpu.MemorySpace.SMEM),
            pl.BlockSpec(memory_space=pltpu.MemorySpace.HBM),
            pl.BlockSpec(memory_space=pltpu.MemorySpace.HBM),
        ],
        out_specs=pl.BlockSpec(memory_space=pltpu.MemorySpace.SMEM),
        out_shape=jax.ShapeDtypeStruct((1, 1), jnp.float32),
        scratch_shapes=[
            pltpu.VMEM((_B, _T, 128), jnp.float32),
            pltpu.VMEM((_B, _T), jnp.float32),
            pltpu.VMEM((_B, _T), jnp.int32),
            pltpu.SemaphoreType.DMA((_NG,)),
            pltpu.SemaphoreType.DMA,
        ],
    )(seqs, reward, baseline, probs, seqs)
    return out[0, 0]


# speedup vs baseline: 1.0999x; 1.0999x over previous
import jax
import jax.numpy as jnp
from jax import lax
from jax.experimental import pallas as pl
from jax.experimental.pallas import tpu as pltpu

_B, _T, _V = 16, 32, 100000
_G = 4
_NG = _B // _G


def _body(s_smem, r_smem, b_smem, probs_hbm, s_vmem, o_ref, buf, pv_ref, sems):
    for b in range(_B):
        for t in range(_T):
            w = pl.multiple_of((s_smem[b, t] >> 7) << 7, 128)
            pltpu.make_async_copy(
                probs_hbm.at[b, t, pl.ds(w, 128)], buf.at[b, t],
                sems.at[b // _G],
            ).start()

    sv = s_vmem[...]
    lane3 = jnp.broadcast_to((sv & 127)[:, :, None], (_B, _T, 128))
    oh = lax.broadcasted_iota(jnp.int32, (_B, _T, 128), 2) == lane3
    seq_len = jnp.sum((sv > 0).astype(jnp.int32), axis=1, keepdims=True) + 1
    tt = lax.broadcasted_iota(jnp.int32, (_B, _T), 1)
    maskv = (tt < seq_len).astype(jnp.float32)

    for g in range(_NG):
        rows = pl.ds(g * _G, _G)
        pltpu.make_async_copy(
            probs_hbm.at[rows, pl.ds(0, _T), pl.ds(0, 128)],
            buf.at[rows], sems.at[g],
        ).wait()
        pv_ref[rows] = jnp.sum(
            jnp.where(oh[g * _G:(g + 1) * _G], buf[g * _G:(g + 1) * _G], 0.0),
            axis=2,
        )

    lrows = -jnp.log(pv_ref[...] + 1e-10) * maskv
    acc = 0.0
    for b in range(_B):
        acc += (r_smem[b] - b_smem[b]) * jnp.sum(lrows[b])
    o_ref[0, 0] = acc / jnp.sum(maskv)


def kernel(reward, baseline, probs, seqs):
    seqs = seqs.astype(jnp.int32)
    out = pl.pallas_call(
        _body,
        in_specs=[
            pl.BlockSpec(memory_space=pltpu.MemorySpace.SMEM),
            pl.BlockSpec(memory_space=pltpu.MemorySpace.SMEM),
            pl.BlockSpec(memory_space=pltpu.MemorySpace.SMEM),
            pl.BlockSpec(memory_space=pltpu.MemorySpace.HBM),
            pl.BlockSpec(memory_space=pltpu.MemorySpace.VMEM),
        ],
        out_specs=pl.BlockSpec(memory_space=pltpu.MemorySpace.SMEM),
        out_shape=jax.ShapeDtypeStruct((1, 1), jnp.float32),
        scratch_shapes=[
            pltpu.VMEM((_B, _T, 128), jnp.float32),
            pltpu.VMEM((_B, _T), jnp.float32),
            pltpu.SemaphoreType.DMA((_NG,)),
        ],
    )(seqs, reward, baseline, probs, seqs)
    return out[0, 0]
